# Initial kernel scaffold; baseline (speedup 1.0000x reference)
import jax
import jax.numpy as jnp
from jax.experimental import pallas as pl


def kernel(phoneme, a1, f2, phoneme_table, f2_table):
    raise NotImplementedError("write your pallas kernel here")


# speedup vs baseline: 1.4268x; 1.4268x over previous
import math

import jax
import jax.numpy as jnp
from jax import lax
from jax.experimental import pallas as pl
from jax.experimental.pallas import tpu as pltpu
from jax.experimental.pallas import tpu_sc as plsc

B, L, C = 4096, 50, 128
SCALE = math.sqrt(C)
NC, NS = 2, 16
NW = NC * NS
BPW = B // NW
GROUP = 4
NGROUPS = BPW // GROUP


def _sc_body(ph_tbl, f2_tbl, ph_idx, f2_idx, a1_in, out_hbm,
             idx_ph_v, idx_f2_v, a1_v, ph_rows, f2_rows, out_buf, gsem):
    wid = lax.axis_index("s") * NC + lax.axis_index("c")
    base = wid * BPW
    iota = lax.iota(jnp.int32, 16)

    def group_body(g, carry):
        b0 = base + g * GROUP
        pltpu.sync_copy(ph_idx.at[pl.ds(b0, GROUP)], idx_ph_v)
        pltpu.sync_copy(f2_idx.at[pl.ds(b0, GROUP)], idx_f2_v)
        pltpu.sync_copy(a1_in.at[pl.ds(b0 * L, GROUP * L)], a1_v)
        for j in range(GROUP):
            b = b0 + j
            cp1 = pltpu.make_async_copy(ph_tbl.at[idx_ph_v.at[j]], ph_rows,
                                        gsem)
            cp1.start()
            cp2 = pltpu.make_async_copy(f2_tbl.at[idx_f2_v.at[j]], f2_rows,
                                        gsem)
            cp2.start()
            cp1.wait()
            cp2.wait()

            def l_body(l, carry2):
                av = plsc.load_gather(a1_v, [jnp.full((16,), j * L,
                                                      jnp.int32) + l])
                for c0 in range(0, C, 16):
                    base_c = (iota + c0) * L
                    idx = base_c + l
                    v = ph_rows[l, pl.ds(c0, 16)] * SCALE
                    plsc.store_scatter(out_buf, [idx], v)
                    w = f2_rows[l, pl.ds(c0, 16)] * SCALE
                    plsc.store_scatter(out_buf, [idx + C * L], w)
                    plsc.store_scatter(out_buf, [idx + 2 * C * L], av)
                return carry2

            lax.fori_loop(0, L, l_body, 0)
            pltpu.sync_copy(out_buf, out_hbm.at[b])
        return carry

    lax.fori_loop(0, NGROUPS, group_body, 0)


def kernel(phoneme, a1, f2, phoneme_table, f2_table):
    mesh = plsc.VectorSubcoreMesh(core_axis_name="c", subcore_axis_name="s")
    f = pl.kernel(
        _sc_body,
        out_type=jax.ShapeDtypeStruct((B, 3 * C * L), jnp.float32),
        mesh=mesh,
        compiler_params=pltpu.CompilerParams(needs_layout_passes=False),
        scratch_types=[
            pltpu.VMEM((GROUP, L), jnp.int32),
            pltpu.VMEM((GROUP, L), jnp.int32),
            pltpu.VMEM((GROUP * L,), jnp.float32),
            pltpu.VMEM((L, C), jnp.float32),
            pltpu.VMEM((L, C), jnp.float32),
            pltpu.VMEM((3 * C * L,), jnp.float32),
            pltpu.SemaphoreType.DMA,
        ],
    )
    out = f(phoneme_table, f2_table, phoneme, f2, a1.reshape(B * L))
    return out.reshape(B, 3 * C, L)


# speedup vs baseline: 1.8010x; 1.2622x over previous
import math

import jax
import jax.numpy as jnp
from jax import lax
from jax.experimental import pallas as pl
from jax.experimental.pallas import tpu as pltpu
from jax.experimental.pallas import tpu_sc as plsc

B, L, C = 4096, 50, 128
SCALE = math.sqrt(C)
NC, NS = 2, 16
NW = NC * NS
BPW = B // NW


def _sc_body(ph_tbl, f2_tbl, ph_idx, f2_idx, a1_in, out_hbm,
             idx_ph_v, idx_f2_v, a1_v,
             ph_rows0, ph_rows1, f2_rows0, f2_rows1, out_buf0, out_buf1,
             gsem0, gsem1, osem0, osem1):
    wid = lax.axis_index("s") * NC + lax.axis_index("c")
    base = wid * BPW
    iota = lax.iota(jnp.int32, 16)
    ph_rows = (ph_rows0, ph_rows1)
    f2_rows = (f2_rows0, f2_rows1)
    out_buf = (out_buf0, out_buf1)
    gsem = (gsem0, gsem1)
    osem = (osem0, osem1)

    pltpu.sync_copy(ph_idx.at[pl.ds(base, BPW)], idx_ph_v)
    pltpu.sync_copy(f2_idx.at[pl.ds(base, BPW)], idx_f2_v)
    pltpu.sync_copy(a1_in.at[pl.ds(base * L, BPW * L)], a1_v)

    def start_gathers(bl, p):
        pltpu.make_async_copy(ph_tbl.at[idx_ph_v.at[bl]], ph_rows[p],
                              gsem[p]).start()
        pltpu.make_async_copy(f2_tbl.at[idx_f2_v.at[bl]], f2_rows[p],
                              gsem[p]).start()

    def wait_gathers(bl, p):
        pltpu.make_async_copy(ph_tbl.at[idx_ph_v.at[bl]], ph_rows[p],
                              gsem[p]).wait()
        pltpu.make_async_copy(f2_tbl.at[idx_f2_v.at[bl]], f2_rows[p],
                              gsem[p]).wait()

    start_gathers(0, 0)

    def pair_body(k, carry):
        for p in range(2):
            bl = k * 2 + p
            b = base + bl

            @pl.when(bl + 1 < BPW)
            def _():
                start_gathers(bl + 1, 1 - p)

            wait_gathers(bl, p)

            @pl.when(k > 0)
            def _():
                pltpu.make_async_copy(out_buf[p], out_hbm.at[b],
                                      osem[p]).wait()

            def l_body(l, carry2):
                av = plsc.load_gather(a1_v, [jnp.full((16,), bl * L,
                                                      jnp.int32) + l])
                for c0 in range(0, C, 16):
                    idx = (iota + c0) * L + l
                    v = ph_rows[p][l, pl.ds(c0, 16)] * SCALE
                    plsc.store_scatter(out_buf[p], [idx], v)
                    w = f2_rows[p][l, pl.ds(c0, 16)] * SCALE
                    plsc.store_scatter(out_buf[p], [idx + C * L], w)
                    plsc.store_scatter(out_buf[p], [idx + 2 * C * L], av)
                return carry2

            lax.fori_loop(0, L, l_body, 0)
            pltpu.make_async_copy(out_buf[p], out_hbm.at[b], osem[p]).start()
        return carry

    lax.fori_loop(0, BPW // 2, pair_body, 0)

    for p in range(2):
        pltpu.make_async_copy(out_buf[p], out_hbm.at[base], osem[p]).wait()


def kernel(phoneme, a1, f2, phoneme_table, f2_table):
    mesh = plsc.VectorSubcoreMesh(core_axis_name="c", subcore_axis_name="s")
    f = pl.kernel(
        _sc_body,
        out_type=jax.ShapeDtypeStruct((B, 3 * C * L), jnp.float32),
        mesh=mesh,
        compiler_params=pltpu.CompilerParams(needs_layout_passes=False),
        scratch_types=[
            pltpu.VMEM((BPW, L), jnp.int32),
            pltpu.VMEM((BPW, L), jnp.int32),
            pltpu.VMEM((BPW * L,), jnp.float32),
            pltpu.VMEM((L, C), jnp.float32),
            pltpu.VMEM((L, C), jnp.float32),
            pltpu.VMEM((L, C), jnp.float32),
            pltpu.VMEM((L, C), jnp.float32),
            pltpu.VMEM((3 * C * L,), jnp.float32),
            pltpu.VMEM((3 * C * L,), jnp.float32),
            pltpu.SemaphoreType.DMA,
            pltpu.SemaphoreType.DMA,
            pltpu.SemaphoreType.DMA,
            pltpu.SemaphoreType.DMA,
        ],
    )
    out = f(phoneme_table, f2_table, phoneme, f2, a1.reshape(B * L))
    return out.reshape(B, 3 * C, L)
